# Initial kernel scaffold; baseline (speedup 1.0000x reference)
import jax
import jax.numpy as jnp
from jax.experimental import pallas as pl


def kernel(x):
    raise NotImplementedError("write your pallas kernel here")


# speedup vs baseline: 1.0068x; 1.0068x over previous
import jax
import jax.numpy as jnp
from jax import lax
from jax.experimental import pallas as pl

_BLK = 256


def _body(x_ref, o_ref):
    j = pl.program_id(1)

    @pl.when(j == 0)
    def _():
        v = x_ref[...]
        row = lax.broadcasted_iota(jnp.int32, v.shape, 1)
        o_ref[...] = jnp.where(row < 3, jnp.float32(-1.0), v)

    @pl.when(j != 0)
    def _():
        o_ref[...] = x_ref[...]


def kernel(x):
    b, r, c = x.shape
    return pl.pallas_call(
        _body,
        grid=(b, r // _BLK),
        in_specs=[pl.BlockSpec((1, _BLK, c), lambda i, j: (i, j, 0))],
        out_specs=pl.BlockSpec((1, _BLK, c), lambda i, j: (i, j, 0)),
        out_shape=jax.ShapeDtypeStruct(x.shape, x.dtype),
    )(x)
